# Initial kernel scaffold; baseline (speedup 1.0000x reference)
import jax
import jax.numpy as jnp
from jax.experimental import pallas as pl


def kernel(words, emits):
    raise NotImplementedError("write your pallas kernel here")


# speedup vs baseline: 4.1421x; 4.1421x over previous
import functools

import jax
import jax.numpy as jnp
from jax import lax
from jax.experimental import pallas as pl
from jax.experimental.pallas import tpu as pltpu
from jax.experimental.pallas import tpu_sc as plsc

_N_TAGS = 64
_NUM_WORKERS = 32
_IDXROW = 128
_CHUNK = 1024
_K = _CHUNK // _IDXROW


@functools.lru_cache(maxsize=None)
def _make_gather(n_tokens: int):
    b_per_w = n_tokens // _NUM_WORKERS
    n_chunks = b_per_w // _CHUNK
    mesh = plsc.VectorSubcoreMesh(core_axis_name="c", subcore_axis_name="s")

    @functools.partial(
        pl.kernel,
        out_type=jax.ShapeDtypeStruct((n_tokens, _N_TAGS), jnp.float32),
        mesh=mesh,
        scratch_types=[
            pltpu.VMEM((_K, _IDXROW), jnp.int32),
            pltpu.VMEM((_CHUNK, _N_TAGS), jnp.float32),
            pltpu.SemaphoreType.DMA,
        ],
        compiler_params=pltpu.CompilerParams(use_tc_tiling_on_sc=False),
    )
    def gather(table_hbm, idx_hbm, out_hbm, idx_v, rows_v, sem):
        wid = lax.axis_index("s") * 2 + lax.axis_index("c")
        base = wid * b_per_w

        def body(i, carry):
            off = pl.multiple_of(base + i * _CHUNK, _CHUNK)
            pltpu.sync_copy(
                idx_hbm.at[pl.ds(pl.multiple_of(off // _IDXROW, _K), _K)], idx_v
            )
            copies = [
                pltpu.async_copy(
                    table_hbm.at[idx_v.at[j]],
                    rows_v.at[pl.ds(j * _IDXROW, _IDXROW)],
                    sem,
                )
                for j in range(_K)
            ]
            for cp in copies:
                cp.wait()
            pltpu.sync_copy(rows_v, out_hbm.at[pl.ds(off, _CHUNK)])
            return carry

        lax.fori_loop(0, n_chunks, body, 0)

    return gather


def kernel(words, emits):
    b, t = words.shape
    n_tags = emits.shape[0]
    n_tokens = b * t
    table = emits.T
    idx = words.reshape(n_tokens // _IDXROW, _IDXROW)
    out = _make_gather(n_tokens)(table, idx)
    return out.reshape(b, t, n_tags)


# speedup vs baseline: 4.2531x; 1.0268x over previous
import functools

import jax
import jax.numpy as jnp
from jax import lax
from jax.experimental import pallas as pl
from jax.experimental.pallas import tpu as pltpu
from jax.experimental.pallas import tpu_sc as plsc

_N_TAGS = 64
_NUM_WORKERS = 32
_IDXROW = 128
_CHUNK = 512
_K = _CHUNK // _IDXROW


@functools.lru_cache(maxsize=None)
def _make_gather(n_tokens: int):
    b_per_w = n_tokens // _NUM_WORKERS
    n_chunks = b_per_w // _CHUNK
    assert n_chunks % 2 == 0
    mesh = plsc.VectorSubcoreMesh(core_axis_name="c", subcore_axis_name="s")

    @functools.partial(
        pl.kernel,
        out_type=jax.ShapeDtypeStruct((n_tokens, _N_TAGS), jnp.float32),
        mesh=mesh,
        scratch_types=[
            pltpu.VMEM((b_per_w // _IDXROW, _IDXROW), jnp.int32),
            pltpu.VMEM((2, _CHUNK, _N_TAGS), jnp.float32),
            pltpu.SemaphoreType.DMA,
            pltpu.SemaphoreType.DMA,
            pltpu.SemaphoreType.DMA,
            pltpu.SemaphoreType.DMA,
        ],
        compiler_params=pltpu.CompilerParams(use_tc_tiling_on_sc=False),
    )
    def gather(table_hbm, idx_hbm, out_hbm, idx_v, rows_v, g0, g1, s0, s1):
        gsem = (g0, g1)
        ssem = (s0, s1)
        wid = lax.axis_index("s") * 2 + lax.axis_index("c")
        base = wid * b_per_w
        pltpu.sync_copy(
            idx_hbm.at[
                pl.ds(pl.multiple_of(base // _IDXROW, b_per_w // _IDXROW),
                      b_per_w // _IDXROW)
            ],
            idx_v,
        )

        def fire_gather(g, p):
            for j in range(_K):
                pltpu.async_copy(
                    table_hbm.at[idx_v.at[g * _K + j]],
                    rows_v.at[p].at[pl.ds(j * _IDXROW, _IDXROW)],
                    gsem[p],
                )

        def wait_gather(p):
            pltpu.make_async_copy(
                table_hbm.at[pl.ds(0, _CHUNK)], rows_v.at[p], gsem[p]
            ).wait()

        def fire_store(g, p):
            off = pl.multiple_of(base + g * _CHUNK, _CHUNK)
            pltpu.async_copy(rows_v.at[p], out_hbm.at[pl.ds(off, _CHUNK)],
                             ssem[p])

        def wait_store(p):
            pltpu.make_async_copy(
                rows_v.at[p], out_hbm.at[pl.ds(0, _CHUNK)], ssem[p]
            ).wait()

        fire_gather(0, 0)

        def outer(i, carry):
            for b in range(2):
                g = i * 2 + b
                p = b
                q = 1 - b

                @pl.when(g + 1 < n_chunks)
                def _():
                    @pl.when(g >= 1)
                    def _():
                        wait_store(q)

                    fire_gather(g + 1, q)

                wait_gather(p)
                fire_store(g, p)
            return carry

        lax.fori_loop(0, n_chunks // 2, outer, 0)
        wait_store(0)
        wait_store(1)

    return gather


def kernel(words, emits):
    b, t = words.shape
    n_tags = emits.shape[0]
    n_tokens = b * t
    table = emits.T
    idx = words.reshape(n_tokens // _IDXROW, _IDXROW)
    out = _make_gather(n_tokens)(table, idx)
    return out.reshape(b, t, n_tags)
